# baseline (speedup 1.0000x reference)
import functools

import jax
import jax.numpy as jnp
from jax import lax
from jax.experimental import pallas as pl
from jax.experimental.pallas import tpu as pltpu
from jax.experimental.pallas import tpu_sc as plsc

D = 64
SCALE = float(D) ** 0.5

R = 4096
T = 200
NC = 2
NS = 16
NW = NC * NS
RW = R // NW
V2 = 500000

_mesh = plsc.VectorSubcoreMesh(core_axis_name="c", subcore_axis_name="s")


@functools.partial(
    pl.kernel,
    mesh=_mesh,
    out_type=jax.ShapeDtypeStruct((T, D, R), jnp.float32),
    scratch_types=[
        pltpu.VMEM((T, RW), jnp.int32),
        pltpu.VMEM((T, RW), jnp.int32),
        pltpu.VMEM((2, RW, 128), jnp.float32),
        pltpu.VMEM((2, D, RW), jnp.float32),
        pltpu.SemaphoreType.DMA((2,)),
        pltpu.SemaphoreType.DMA((2,)),
    ],
    compiler_params=pltpu.CompilerParams(
        use_tc_tiling_on_sc=True, needs_layout_passes=False
    ),
)
def _embed(idxt_hbm, table2_hbm, out_hbm, idx_v, idx2_v, rows_v, ot_v, gsem, wsem):
    wid = lax.axis_index("s") * NC + lax.axis_index("c")
    r0 = wid * RW

    pltpu.sync_copy(idxt_hbm.at[:, pl.ds(r0, RW)], idx_v)

    def halve(t, carry):
        for g in range(RW // 16):
            sl = pl.ds(g * 16, 16)
            idx2_v[t, sl] = lax.shift_right_logical(idx_v[t, sl], 1)
        return carry

    lax.fori_loop(0, T, halve, 0, unroll=4)

    def fire_gather(t, b):
        pltpu.async_copy(table2_hbm.at[idx2_v.at[t]], rows_v.at[b], gsem.at[b])

    def wait_gather(t, b):
        pltpu.make_async_copy(
            table2_hbm.at[idx2_v.at[t]], rows_v.at[b], gsem.at[b]
        ).wait()

    def fire_write(t, b):
        pltpu.async_copy(ot_v.at[b], out_hbm.at[t, :, pl.ds(r0, RW)], wsem.at[b])

    def wait_write(t, b):
        pltpu.make_async_copy(
            ot_v.at[b], out_hbm.at[t, :, pl.ds(r0, RW)], wsem.at[b]
        ).wait()

    iota = lax.iota(jnp.int32, 16)
    civ = [iota + c * 16 for c in range(D // 16)]

    def take16(x, rot):
        return lax.gather(
            x,
            rot[:, None],
            dimension_numbers=lax.GatherDimensionNumbers(
                offset_dims=(), collapsed_slice_dims=(0,), start_index_map=(0,)
            ),
            slice_sizes=(1,),
            mode=lax.GatherScatterMode.PROMISE_IN_BOUNDS,
        )

    def compute(t, b):
        for jg in range(RW // 16):
            sl = pl.ds(jg * 16, 16)
            idx16 = idx_v[t, sl]
            s = jnp.where(idx16 != 0, SCALE, 0.0).astype(jnp.float32)
            par = (idx16 & 1) * D

            def kbody(k, carry):
                rot = (iota + k) & 15
                rows = rot + jg * 16
                s_k = take16(s, rot)
                p_k = take16(par, rot)
                for c in range(D // 16):
                    colv = p_k + civ[c]
                    vals = plsc.load_gather(rows_v.at[b], [rows, colv])
                    plsc.store_scatter(ot_v.at[b], [civ[c], rows], vals * s_k)
                return carry

            lax.fori_loop(0, 16, kbody, 0, unroll=4)

    def step(k, carry):
        for u in range(2):
            t = k * 2 + u
            b = u
            wait_gather(t, b)

            @pl.when(t >= 2)
            def _():
                wait_write(t - 2, b)

            compute(t, b)
            fire_write(t, b)

            @pl.when(t + 2 < T)
            def _():
                fire_gather(t + 2, b)

        return carry

    fire_gather(0, 0)
    fire_gather(1, 1)
    lax.fori_loop(0, T // 2, step, 0)
    wait_write(T - 2, 0)
    wait_write(T - 1, 1)


def kernel(input, lookup_table):
    idxt = input.astype(jnp.int32).T
    table2 = lookup_table.reshape(V2, 2 * D)
    out = _embed(idxt, table2)
    return out.transpose(2, 0, 1)


# speedup vs baseline: 1.1485x; 1.1485x over previous
import functools

import jax
import jax.numpy as jnp
from jax import lax
from jax.experimental import pallas as pl
from jax.experimental.pallas import tpu as pltpu
from jax.experimental.pallas import tpu_sc as plsc

D = 64
SCALE = float(D) ** 0.5

R = 4096
T = 200
NC = 2
NS = 16
NW = NC * NS
RW = R // NW
V2 = 500000
NG = 4
NO = 2

_mesh = plsc.VectorSubcoreMesh(core_axis_name="c", subcore_axis_name="s")


@functools.partial(
    pl.kernel,
    mesh=_mesh,
    out_type=jax.ShapeDtypeStruct((T, D, R), jnp.float32),
    scratch_types=[
        pltpu.VMEM((T, RW), jnp.int32),
        pltpu.VMEM((NG, RW), jnp.int32),
        pltpu.VMEM((NG, RW, 128), jnp.float32),
        pltpu.VMEM((NO, D, RW), jnp.float32),
        pltpu.SemaphoreType.DMA((NG,)),
        pltpu.SemaphoreType.DMA((NO,)),
    ],
    compiler_params=pltpu.CompilerParams(
        use_tc_tiling_on_sc=True, needs_layout_passes=False
    ),
)
def _embed(idxt_hbm, table2_hbm, out_hbm, idx_v, i2r_v, rows_v, ot_v, gsem, wsem):
    wid = lax.axis_index("s") * NC + lax.axis_index("c")
    r0 = wid * RW

    pltpu.sync_copy(idxt_hbm.at[:, pl.ds(r0, RW)], idx_v)

    def prep_fire_gather(t, b):
        for g in range(RW // 16):
            sl = pl.ds(g * 16, 16)
            i2r_v[b, sl] = lax.shift_right_logical(idx_v[t, sl], 1)
        pltpu.async_copy(table2_hbm.at[i2r_v.at[b]], rows_v.at[b], gsem.at[b])

    def wait_gather(b):
        pltpu.make_async_copy(
            table2_hbm.at[i2r_v.at[b]], rows_v.at[b], gsem.at[b]
        ).wait()

    def fire_write(t, ob):
        pltpu.async_copy(ot_v.at[ob], out_hbm.at[t, :, pl.ds(r0, RW)], wsem.at[ob])

    def wait_write(t, ob):
        pltpu.make_async_copy(
            ot_v.at[ob], out_hbm.at[t, :, pl.ds(r0, RW)], wsem.at[ob]
        ).wait()

    iota = lax.iota(jnp.int32, 16)
    civ = [iota + c * 16 for c in range(D // 16)]

    def take16(x, rot):
        return lax.gather(
            x,
            rot[:, None],
            dimension_numbers=lax.GatherDimensionNumbers(
                offset_dims=(), collapsed_slice_dims=(0,), start_index_map=(0,)
            ),
            slice_sizes=(1,),
            mode=lax.GatherScatterMode.PROMISE_IN_BOUNDS,
        )

    def compute(t, b, ob):
        for jg in range(RW // 16):
            sl = pl.ds(jg * 16, 16)
            idx16 = idx_v[t, sl]
            s = jnp.where(idx16 != 0, SCALE, 0.0).astype(jnp.float32)
            par = (idx16 & 1) * D

            def kbody(k, carry):
                rot = (iota + k) & 15
                rows = rot + jg * 16
                s_k = take16(s, rot)
                p_k = take16(par, rot)
                for c in range(D // 16):
                    colv = p_k + civ[c]
                    vals = plsc.load_gather(rows_v.at[b], [rows, colv])
                    plsc.store_scatter(ot_v.at[ob], [civ[c], rows], vals * s_k)
                return carry

            lax.fori_loop(0, 16, kbody, 0, unroll=2)

    for b in range(NG):
        prep_fire_gather(b, b)

    def step(k, carry):
        for u in range(NG):
            t = k * NG + u
            b = u
            ob = u % NO
            wait_gather(b)

            @pl.when(t >= NO)
            def _():
                wait_write(t - NO, ob)

            compute(t, b, ob)
            fire_write(t, ob)

            @pl.when(t + NG < T)
            def _():
                prep_fire_gather(t + NG, b)

        return carry

    lax.fori_loop(0, T // NG, step, 0)
    wait_write(T - 2, 0)
    wait_write(T - 1, 1)


def kernel(input, lookup_table):
    idxt = input.astype(jnp.int32).T
    table2 = lookup_table.reshape(V2, 2 * D)
    out = _embed(idxt, table2)
    return out.transpose(2, 0, 1)


# speedup vs baseline: 1.1965x; 1.0418x over previous
import functools

import jax
import jax.numpy as jnp
from jax import lax
from jax.experimental import pallas as pl
from jax.experimental.pallas import tpu as pltpu
from jax.experimental.pallas import tpu_sc as plsc

D = 64
SCALE = float(D) ** 0.5

R = 4096
T = 200
NC = 2
NS = 16
NW = NC * NS
ROWS_W = R // NW
G0 = 128
G1 = T - G0
NBUF = 4

_mesh = plsc.VectorSubcoreMesh(core_axis_name="c", subcore_axis_name="s")


@functools.partial(
    pl.kernel,
    mesh=_mesh,
    out_type=jax.ShapeDtypeStruct((R, T, D), jnp.float32),
    scratch_types=[
        pltpu.VMEM((ROWS_W, T), jnp.int32),
        pltpu.VMEM((NBUF, T, D), jnp.float32),
        pltpu.SemaphoreType.DMA((NBUF,)),
        pltpu.SemaphoreType.DMA((NBUF,)),
    ],
    compiler_params=pltpu.CompilerParams(use_tc_tiling_on_sc=False),
)
def _embed(idx_hbm, table_hbm, out_hbm, idx_v, rows_v, gsem, osem):
    wid = lax.axis_index("s") * NC + lax.axis_index("c")
    row0 = wid * ROWS_W

    pltpu.sync_copy(idx_hbm.at[pl.ds(row0, ROWS_W)], idx_v)

    def fire_gather(r, b):
        pltpu.async_copy(
            table_hbm.at[idx_v.at[r, pl.ds(0, G0)]],
            rows_v.at[b, pl.ds(0, G0)],
            gsem.at[b],
        )
        pltpu.async_copy(
            table_hbm.at[idx_v.at[r, pl.ds(G0, G1)]],
            rows_v.at[b, pl.ds(G0, G1)],
            gsem.at[b],
        )

    def wait_gather(r, b):
        pltpu.make_async_copy(
            table_hbm.at[idx_v.at[r, pl.ds(0, G0)]],
            rows_v.at[b, pl.ds(0, G0)],
            gsem.at[b],
        ).wait()
        pltpu.make_async_copy(
            table_hbm.at[idx_v.at[r, pl.ds(G0, G1)]],
            rows_v.at[b, pl.ds(G0, G1)],
            gsem.at[b],
        ).wait()

    def fire_scatter(r, b):
        pltpu.async_copy(rows_v.at[b], out_hbm.at[row0 + r], osem.at[b])

    def wait_scatter(r, b):
        pltpu.make_async_copy(rows_v.at[b], out_hbm.at[row0 + r], osem.at[b]).wait()

    def scale16(b, s, tok0, n):
        for j in range(n):
            sj = s[16 - n + j]
            t = tok0 + j
            for c in range(D // 16):
                sl = pl.ds(c * 16, 16)
                rows_v[b, t, sl] = rows_v[b, t, sl] * sj

    def compute(r, b):
        def grp_body(g, c2):
            idx16 = idx_v[r, pl.ds(g * 16, 16)]
            s = jnp.where(idx16 != 0, SCALE, 0.0).astype(jnp.float32)
            for j in range(16):
                sj = s[j]
                t = g * 16 + j
                for c in range(D // 16):
                    sl = pl.ds(c * 16, 16)
                    rows_v[b, t, sl] = rows_v[b, t, sl] * sj
            return c2

        lax.fori_loop(0, (T // 16), grp_body, 0, unroll=2)
        idx16 = idx_v[r, pl.ds(T - 16, 16)]
        s = jnp.where(idx16 != 0, SCALE, 0.0).astype(jnp.float32)
        scale16(b, s, (T // 16) * 16, T - (T // 16) * 16)

    for b in range(NBUF - 1):
        fire_gather(b, b)

    def step(k, carry):
        for u in range(NBUF):
            i = k * NBUF + u
            b = u
            wait_gather(i, b)
            compute(i, b)
            fire_scatter(i, b)
            nxt = i + (NBUF - 1)
            bn = (u + NBUF - 1) % NBUF

            @pl.when(nxt < ROWS_W)
            def _():
                @pl.when(i >= 1)
                def _():
                    wait_scatter(i - 1, bn)

                fire_gather(nxt, bn)

        return carry

    lax.fori_loop(0, ROWS_W // NBUF, step, 0)

    for u in range(NBUF):
        r = ROWS_W - NBUF + u
        wait_scatter(r, r % NBUF)


def kernel(input, lookup_table):
    return _embed(input.astype(jnp.int32), lookup_table)
